# baseline (speedup 1.0000x reference)
import functools

import jax
import jax.numpy as jnp
from jax import lax
from jax.experimental import pallas as pl
from jax.experimental.pallas import tpu as pltpu
from jax.experimental.pallas import tpu_sc as plsc

B = 4
T = 8192
D = 768
PAD = 2

_info = plsc.get_sparse_core_info()
NC = _info.num_cores
NS = _info.num_subcores
NW = NC * NS
ROWS_PER_W = T // NW
CH = 32
NBUF = 4
NCHUNK = ROWS_PER_W // CH

_mesh = plsc.VectorSubcoreMesh(core_axis_name="c", subcore_axis_name="s")


@functools.partial(
    pl.kernel,
    mesh=_mesh,
    out_type=jax.ShapeDtypeStruct((B * T * D,), jnp.float32),
    scratch_types=(
        [pltpu.VMEM((CH * D,), jnp.float32) for _ in range(NBUF)]
        + [pltpu.SemaphoreType.DMA for _ in range(2 * NBUF)]
    ),
)
def _pos_embed(w_hbm, out_hbm, *scratch):
    bufs = scratch[:NBUF]
    lsems = scratch[NBUF : 2 * NBUF]
    ssems = scratch[2 * NBUF :]
    wid = lax.axis_index("s") * NC + lax.axis_index("c")
    base = wid * ROWS_PER_W

    def load(i):
        off = (PAD + base + i * CH) * D
        return pltpu.async_copy(
            w_hbm.at[pl.ds(off, CH * D)], bufs[i % NBUF], lsems[i % NBUF]
        )

    def fire_stores(i):
        return [
            pltpu.async_copy(
                bufs[i % NBUF],
                out_hbm.at[pl.ds((b * T + base + i * CH) * D, CH * D)],
                ssems[i % NBUF],
            )
            for b in range(B)
        ]

    loads = {i: load(i) for i in range(min(NBUF, NCHUNK))}
    stores = {}
    for i in range(NCHUNK):
        loads[i].wait()
        stores[i] = fire_stores(i)
        if i + NBUF < NCHUNK:
            for h in stores[i]:
                h.wait()
            loads[i + NBUF] = load(i + NBUF)
    for i in range(max(0, NCHUNK - NBUF), NCHUNK):
        for h in stores[i]:
            h.wait()


def kernel(input, weights):
    del input
    flat = _pos_embed(weights.reshape(-1))
    return flat.reshape(B, T, D)


# speedup vs baseline: 3.0838x; 3.0838x over previous
import functools

import jax
import jax.numpy as jnp
from jax import lax
from jax.experimental import pallas as pl
from jax.experimental.pallas import tpu as pltpu
from jax.experimental.pallas import tpu_sc as plsc

B = 4
T = 8192
D = 768
PAD = 2
NROWS = T + PAD

_info = plsc.get_sparse_core_info()
NC = _info.num_cores
NS = _info.num_subcores
L = _info.num_lanes
NW = NC * NS
ROWS_PER_W = T // NW
CH = 64
NBUF = 2
NCHUNK = ROWS_PER_W // CH

_mesh = plsc.VectorSubcoreMesh(core_axis_name="c", subcore_axis_name="s")


@functools.partial(
    pl.kernel,
    mesh=_mesh,
    out_type=jax.ShapeDtypeStruct((B, T, D), jnp.float32),
    scratch_types=(
        [pltpu.VMEM((CH, D), jnp.float32) for _ in range(NBUF)]
        + [pltpu.VMEM((ROWS_PER_W,), jnp.int32)]
        + [pltpu.SemaphoreType.DMA for _ in range(2 * NBUF)]
    ),
)
def _pos_embed(w_hbm, out_hbm, *scratch):
    bufs = scratch[:NBUF]
    idx = scratch[NBUF]
    lsems = scratch[NBUF + 1 : NBUF + 1 + NBUF]
    ssems = scratch[NBUF + 1 + NBUF :]
    wid = lax.axis_index("s") * NC + lax.axis_index("c")
    base = wid * ROWS_PER_W

    lane = lax.iota(jnp.int32, L)
    for j in range(ROWS_PER_W // L):
        idx[pl.ds(j * L, L)] = lane + (PAD + base + j * L)

    def start_load(i):
        return pltpu.async_copy(
            w_hbm.at[idx.at[pl.ds(i * CH, CH)]], bufs[i % NBUF], lsems[i % NBUF]
        )

    def fire_stores(i):
        return [
            pltpu.async_copy(
                bufs[i % NBUF],
                out_hbm.at[b, pl.ds(base + i * CH, CH)],
                ssems[i % NBUF],
            )
            for b in range(B)
        ]

    loads = {i: start_load(i) for i in range(min(NBUF, NCHUNK))}
    stores = {}
    for i in range(NCHUNK):
        loads[i].wait()
        stores[i] = fire_stores(i)
        if i + NBUF < NCHUNK:
            for h in stores[i]:
                h.wait()
            loads[i + NBUF] = start_load(i + NBUF)
    for i in range(max(0, NCHUNK - NBUF), NCHUNK):
        for h in stores[i]:
            h.wait()


def kernel(input, weights):
    del input
    return _pos_embed(weights)
